# Initial kernel scaffold; baseline (speedup 1.0000x reference)
import jax
import jax.numpy as jnp
from jax.experimental import pallas as pl


def kernel(positions, pe):
    raise NotImplementedError("write your pallas kernel here")


# speedup vs baseline: 2.1887x; 2.1887x over previous
import functools
import jax
import jax.numpy as jnp
from jax import lax
from jax.experimental import pallas as pl
from jax.experimental.pallas import tpu as pltpu, tpu_sc as plsc

_OUTPUT_DIM = 1024
_CHUNK = 64


def _make_gather(total_rows, dim):
    info = plsc.get_sparse_core_info()
    nc, ns = info.num_cores, info.num_subcores
    nw = nc * ns
    assert total_rows % (nw * _CHUNK) == 0
    rows_per_w = total_rows // nw
    iters = rows_per_w // _CHUNK
    mesh = plsc.VectorSubcoreMesh(core_axis_name="c", subcore_axis_name="s")

    @functools.partial(
        pl.kernel,
        mesh=mesh,
        out_type=jax.ShapeDtypeStruct((total_rows, dim), jnp.float32),
        scratch_types=[
            pltpu.VMEM((rows_per_w,), jnp.int32),
            pltpu.VMEM((_CHUNK, dim), jnp.float32),
            pltpu.SemaphoreType.DMA,
        ],
    )
    def k(pos_hbm, table_hbm, out_hbm, idx_v, rows_v, sem):
        wid = lax.axis_index("s") * nc + lax.axis_index("c")
        base = wid * rows_per_w
        pltpu.sync_copy(pos_hbm.at[pl.ds(base, rows_per_w)], idx_v)

        def body(g, _):
            off = g * _CHUNK
            pltpu.async_copy(
                table_hbm.at[idx_v.at[pl.ds(off, _CHUNK)]], rows_v, sem
            ).wait()
            pltpu.sync_copy(rows_v, out_hbm.at[pl.ds(base + off, _CHUNK)])
            return 0

        lax.fori_loop(0, iters, body, 0)

    return k


def kernel(positions, pe):
    if positions.ndim == 1:
        positions = positions[None, :]
    batch, seq = positions.shape
    flat = positions.reshape(-1)
    out = _make_gather(batch * seq, pe.shape[1])(flat, pe)
    return out.reshape(batch, seq, pe.shape[1])


# speedup vs baseline: 2.3851x; 1.0897x over previous
import functools
import jax
import jax.numpy as jnp
from jax import lax
from jax.experimental import pallas as pl
from jax.experimental.pallas import tpu as pltpu, tpu_sc as plsc

_CHUNK = 32


def _make_gather(total_rows, dim):
    info = plsc.get_sparse_core_info()
    nc, ns = info.num_cores, info.num_subcores
    nw = nc * ns
    assert total_rows % (nw * 2 * _CHUNK) == 0
    rows_per_w = total_rows // nw
    iters = rows_per_w // _CHUNK
    mesh = plsc.VectorSubcoreMesh(core_axis_name="c", subcore_axis_name="s")

    @functools.partial(
        pl.kernel,
        mesh=mesh,
        out_type=jax.ShapeDtypeStruct((total_rows, dim), jnp.float32),
        scratch_types=[
            pltpu.VMEM((rows_per_w,), jnp.int32),
            pltpu.VMEM((_CHUNK, dim), jnp.float32),
            pltpu.VMEM((_CHUNK, dim), jnp.float32),
            pltpu.SemaphoreType.DMA,
            pltpu.SemaphoreType.DMA,
        ],
    )
    def k(pos_hbm, table_hbm, out_hbm, idx_v, buf0, buf1, sem0, sem1):
        wid = lax.axis_index("s") * nc + lax.axis_index("c")
        base = wid * rows_per_w
        pltpu.sync_copy(pos_hbm.at[pl.ds(base, rows_per_w)], idx_v)

        def gather(g, buf, sem):
            pltpu.async_copy(
                table_hbm.at[idx_v.at[pl.ds(g * _CHUNK, _CHUNK)]], buf, sem
            )

        def wait_gather(buf, sem):
            pltpu.make_async_copy(table_hbm.at[pl.ds(0, _CHUNK)], buf, sem).wait()

        def write(g, buf):
            pltpu.sync_copy(buf, out_hbm.at[pl.ds(base + g * _CHUNK, _CHUNK)])

        gather(0, buf0, sem0)
        gather(1, buf1, sem1)

        def body(s, _):
            g = 2 * s
            wait_gather(buf0, sem0)
            write(g, buf0)
            gather(g + 2, buf0, sem0)
            wait_gather(buf1, sem1)
            write(g + 1, buf1)
            gather(g + 3, buf1, sem1)
            return 0

        lax.fori_loop(0, iters // 2 - 1, body, 0)
        wait_gather(buf0, sem0)
        write(iters - 2, buf0)
        wait_gather(buf1, sem1)
        write(iters - 1, buf1)

    return k


def kernel(positions, pe):
    if positions.ndim == 1:
        positions = positions[None, :]
    batch, seq = positions.shape
    flat = positions.reshape(-1)
    out = _make_gather(batch * seq, pe.shape[1])(flat, pe)
    return out.reshape(batch, seq, pe.shape[1])
